# baseline (speedup 1.0000x reference)
import functools

import jax
import jax.numpy as jnp
from jax import lax
from jax.experimental import pallas as pl
from jax.experimental.pallas import tpu as pltpu
from jax.experimental.pallas import tpu_sc as plsc

VOCAB = 1000000
EMBED = 128
BATCH = 4096
HALF = BATCH // 2

_NC = 2
_NS = 16
_NW = _NC * _NS

_mesh = plsc.VectorSubcoreMesh(
    core_axis_name="c", subcore_axis_name="s", num_cores=_NC, num_subcores=_NS
)


def _make_sc_gather(n_rows_a, n_rows_b):
    bpw_a = n_rows_a // _NW
    bpw_b = n_rows_b // _NW

    @functools.partial(
        pl.kernel,
        out_type=(
            jax.ShapeDtypeStruct((n_rows_a, EMBED), jnp.float32),
            jax.ShapeDtypeStruct((n_rows_b, EMBED), jnp.float32),
        ),
        mesh=_mesh,
        scratch_types=[
            pltpu.VMEM((bpw_a,), jnp.int32),
            pltpu.VMEM((bpw_b,), jnp.int32),
            pltpu.VMEM((bpw_a, EMBED), jnp.float32),
            pltpu.VMEM((bpw_b, EMBED), jnp.float32),
            pltpu.SemaphoreType.DMA,
            pltpu.SemaphoreType.DMA,
        ],
    )
    def sc_gather(idx_a_hbm, idx_b_hbm, tab_a_hbm, tab_b_hbm,
                  emb_a_hbm, emb_b_hbm,
                  idx_a_v, idx_b_v, rows_a_v, rows_b_v, sem_a, sem_b):
        wid = lax.axis_index("s") * _NC + lax.axis_index("c")
        base_a = wid * bpw_a
        base_b = wid * bpw_b
        pltpu.sync_copy(idx_a_hbm.at[pl.ds(base_a, bpw_a)], idx_a_v)
        pltpu.sync_copy(idx_b_hbm.at[pl.ds(base_b, bpw_b)], idx_b_v)
        ga = pltpu.async_copy(tab_a_hbm.at[idx_a_v], rows_a_v, sem_a)
        gb = pltpu.async_copy(tab_b_hbm.at[idx_b_v], rows_b_v, sem_b)
        ga.wait()
        wa = pltpu.async_copy(rows_a_v, emb_a_hbm.at[pl.ds(base_a, bpw_a)], sem_a)
        gb.wait()
        wb = pltpu.async_copy(rows_b_v, emb_b_hbm.at[pl.ds(base_b, bpw_b)], sem_b)
        wa.wait()
        wb.wait()

    return sc_gather


_sc_gather_ctx_tgt0 = _make_sc_gather(BATCH, HALF)

_BPW1 = HALF // _NW


@functools.partial(
    pl.kernel,
    out_type=jax.ShapeDtypeStruct((HALF, EMBED), jnp.float32),
    mesh=_mesh,
    scratch_types=[
        pltpu.VMEM((_BPW1,), jnp.int32),
        pltpu.VMEM((_BPW1, EMBED), jnp.float32),
        pltpu.SemaphoreType.DMA,
    ],
)
def _sc_gather_tgt1(idx_hbm, tab_hbm, emb_hbm, idx_v, rows_v, sem):
    wid = lax.axis_index("s") * _NC + lax.axis_index("c")
    base = wid * _BPW1
    pltpu.sync_copy(idx_hbm.at[pl.ds(base, _BPW1)], idx_v)
    pltpu.async_copy(tab_hbm.at[idx_v], rows_v, sem).wait()
    pltpu.sync_copy(rows_v, emb_hbm.at[pl.ds(base, _BPW1)])

_BM = 512


def _mm_body0(a_ref, b_ref, o_ref):
    o_ref[...] = lax.dot_general(
        a_ref[...].astype(jnp.bfloat16), b_ref[...].astype(jnp.bfloat16),
        dimension_numbers=(((1,), (1,)), ((), ())),
        preferred_element_type=jnp.float32,
    )


def _mm_body1(a_ref, b_ref, prev_ref, o_ref):
    del prev_ref
    o_ref[...] = lax.dot_general(
        a_ref[...].astype(jnp.bfloat16), b_ref[...].astype(jnp.bfloat16),
        dimension_numbers=(((1,), (1,)), ((), ())),
        preferred_element_type=jnp.float32,
    )


_matmul0 = pl.pallas_call(
    _mm_body0,
    grid=(HALF // _BM,),
    in_specs=[
        pl.BlockSpec((_BM, EMBED), lambda i: (i, 0)),
        pl.BlockSpec((BATCH, EMBED), lambda i: (0, 0)),
    ],
    out_specs=pl.BlockSpec((_BM, BATCH), lambda i: (i, 0)),
    out_shape=jax.ShapeDtypeStruct((BATCH, BATCH), jnp.float32),
)

_matmul1 = pl.pallas_call(
    _mm_body1,
    grid=(HALF // _BM,),
    in_specs=[
        pl.BlockSpec((_BM, EMBED), lambda i: (i, 0)),
        pl.BlockSpec((BATCH, EMBED), lambda i: (0, 0)),
        pl.BlockSpec(memory_space=pl.ANY),
    ],
    out_specs=pl.BlockSpec((_BM, BATCH), lambda i: (i + HALF // _BM, 0)),
    out_shape=jax.ShapeDtypeStruct((BATCH, BATCH), jnp.float32),
    input_output_aliases={2: 0},
)


def kernel(target, context, in_table, out_table):
    target = target.astype(jnp.int32)
    context = context.astype(jnp.int32)
    out_embeds, in_emb0 = _sc_gather_ctx_tgt0(
        context, target[:HALF], out_table, in_table)
    in_emb1 = _sc_gather_tgt1(target[HALF:], in_table)
    scores0 = _matmul0(in_emb0, out_embeds)
    return _matmul1(in_emb1, out_embeds, scores0)


# speedup vs baseline: 1.0803x; 1.0803x over previous
import functools

import jax
import jax.numpy as jnp
from jax import lax
from jax.experimental import pallas as pl
from jax.experimental.pallas import tpu as pltpu
from jax.experimental.pallas import tpu_sc as plsc

VOCAB = 1000000
EMBED = 128
BATCH = 4096

_NC = 2
_NS = 16
_NW = _NC * _NS
_BPW = BATCH // _NW
_CH = _BPW // 2

_mesh = plsc.VectorSubcoreMesh(
    core_axis_name="c", subcore_axis_name="s", num_cores=_NC, num_subcores=_NS
)


@functools.partial(
    pl.kernel,
    out_type=(
        jax.ShapeDtypeStruct((BATCH, EMBED), jnp.float32),
        jax.ShapeDtypeStruct((BATCH, EMBED), jnp.float32),
    ),
    mesh=_mesh,
    scratch_types=[
        pltpu.VMEM((_BPW,), jnp.int32),
        pltpu.VMEM((_BPW,), jnp.int32),
        pltpu.VMEM((_BPW, EMBED), jnp.float32),
        pltpu.VMEM((_BPW, EMBED), jnp.float32),
        pltpu.SemaphoreType.DMA,
        pltpu.SemaphoreType.DMA,
        pltpu.SemaphoreType.DMA,
        pltpu.SemaphoreType.DMA,
        pltpu.SemaphoreType.DMA,
        pltpu.SemaphoreType.DMA,
    ],
)
def _sc_gather(target_hbm, context_hbm, in_tab_hbm, out_tab_hbm,
               in_emb_hbm, out_emb_hbm,
               tgt_idx_v, ctx_idx_v, in_rows_v, out_rows_v,
               sem_ia, sem_ib, sem_a0, sem_a1, sem_b0, sem_b1):
    wid = lax.axis_index("s") * _NC + lax.axis_index("c")
    base = wid * _BPW
    ia = pltpu.async_copy(target_hbm.at[pl.ds(base, _BPW)], tgt_idx_v, sem_ia)
    ib = pltpu.async_copy(context_hbm.at[pl.ds(base, _BPW)], ctx_idx_v, sem_ib)
    ia.wait()
    ga0 = pltpu.async_copy(in_tab_hbm.at[tgt_idx_v.at[pl.ds(0, _CH)]],
                           in_rows_v.at[pl.ds(0, _CH)], sem_a0)
    ga1 = pltpu.async_copy(in_tab_hbm.at[tgt_idx_v.at[pl.ds(_CH, _CH)]],
                           in_rows_v.at[pl.ds(_CH, _CH)], sem_a1)
    ib.wait()
    gb0 = pltpu.async_copy(out_tab_hbm.at[ctx_idx_v.at[pl.ds(0, _CH)]],
                           out_rows_v.at[pl.ds(0, _CH)], sem_b0)
    gb1 = pltpu.async_copy(out_tab_hbm.at[ctx_idx_v.at[pl.ds(_CH, _CH)]],
                           out_rows_v.at[pl.ds(_CH, _CH)], sem_b1)
    ga0.wait()
    wa0 = pltpu.async_copy(in_rows_v.at[pl.ds(0, _CH)],
                           in_emb_hbm.at[pl.ds(base, _CH)], sem_a0)
    ga1.wait()
    wa1 = pltpu.async_copy(in_rows_v.at[pl.ds(_CH, _CH)],
                           in_emb_hbm.at[pl.ds(base + _CH, _CH)], sem_a1)
    gb0.wait()
    wb0 = pltpu.async_copy(out_rows_v.at[pl.ds(0, _CH)],
                           out_emb_hbm.at[pl.ds(base, _CH)], sem_b0)
    gb1.wait()
    wb1 = pltpu.async_copy(out_rows_v.at[pl.ds(_CH, _CH)],
                           out_emb_hbm.at[pl.ds(base + _CH, _CH)], sem_b1)
    wa0.wait()
    wa1.wait()
    wb0.wait()
    wb1.wait()


_BM = 512


def _mm_body(a_ref, b_ref, o_ref):
    o_ref[...] = lax.dot_general(
        a_ref[...].astype(jnp.bfloat16), b_ref[...].astype(jnp.bfloat16),
        dimension_numbers=(((1,), (1,)), ((), ())),
        preferred_element_type=jnp.float32,
    )


_matmul = pl.pallas_call(
    _mm_body,
    grid=(BATCH // _BM,),
    in_specs=[
        pl.BlockSpec((_BM, EMBED), lambda i: (i, 0)),
        pl.BlockSpec((BATCH, EMBED), lambda i: (0, 0)),
    ],
    out_specs=pl.BlockSpec((_BM, BATCH), lambda i: (i, 0)),
    out_shape=jax.ShapeDtypeStruct((BATCH, BATCH), jnp.float32),
)


def kernel(target, context, in_table, out_table):
    target = target.astype(jnp.int32)
    context = context.astype(jnp.int32)
    in_embeds, out_embeds = _sc_gather(target, context, in_table, out_table)
    return _matmul(in_embeds, out_embeds)
